# Initial kernel scaffold; baseline (speedup 1.0000x reference)
import jax
import jax.numpy as jnp
from jax.experimental import pallas as pl


def kernel(x, edge_index, edge_attr, batch, params):
    raise NotImplementedError("write your pallas kernel here")


# speedup vs baseline: 1.0265x; 1.0265x over previous
import functools

import jax
import jax.numpy as jnp
from jax import lax
from jax.experimental import pallas as pl
from jax.experimental.pallas import tpu as pltpu
from jax.experimental.pallas import tpu_sc as plsc

_HID = 32
_NUM_GRAPHS = 64


def _sc_gather(table, idx, chunk):
    info = plsc.get_sparse_core_info()
    nc, ns = info.num_cores, info.num_subcores
    nw = nc * ns
    n, d = table.shape
    table128 = jnp.pad(table, ((0, 0), (0, 128 - d)))
    e = idx.shape[0]
    b_w = e // nw
    n_ch = b_w // chunk
    mesh = plsc.VectorSubcoreMesh(core_axis_name="c", subcore_axis_name="s")

    @functools.partial(
        pl.kernel,
        mesh=mesh,
        out_type=jax.ShapeDtypeStruct((e, 128), jnp.float32),
        scratch_types=[
            pltpu.VMEM((chunk,), jnp.int32),
            pltpu.VMEM((chunk, 128), jnp.float32),
            pltpu.SemaphoreType.DMA,
        ],
    )
    def gather_kernel(table_hbm, idx_hbm, out_hbm, idx_v, rows_v, sem):
        wid = lax.axis_index("s") * nc + lax.axis_index("c")
        base = wid * b_w
        for j in range(n_ch):
            off = base + j * chunk
            pltpu.sync_copy(idx_hbm.at[pl.ds(off, chunk)], idx_v)
            pltpu.async_copy(table_hbm.at[idx_v], rows_v, sem).wait()
            pltpu.sync_copy(rows_v, out_hbm.at[pl.ds(off, chunk)])

    return gather_kernel(table128, idx)


def _softplus(x):
    return jnp.maximum(x, 0.0) + jnp.log1p(jnp.exp(-jnp.abs(x)))


def _act(x, kind):
    if kind == "relu":
        return jnp.maximum(x, 0.0)
    if kind == "softplus":
        return _softplus(x)
    return x


def _mlp_body(*refs, n_in, acts, widths):
    in_refs = refs[:n_in]
    w_refs = refs[n_in:-1]
    o_ref = refs[-1]
    parts = []
    for r, w in zip(in_refs, widths):
        v = r[...]
        if w is not None and w < v.shape[1]:
            v = v[:, :w]
        parts.append(v)
    h = parts[0] if n_in == 1 else jnp.concatenate(parts, axis=1)
    n_layer = len(w_refs) // 2
    for li in range(n_layer):
        w = w_refs[2 * li][...]
        b = w_refs[2 * li + 1][...]
        h = jnp.dot(h, w, preferred_element_type=jnp.float32) + b
        h = _act(h, acts[li])
    o_ref[...] = h


def _mlp(inputs, layers, acts, tile, m=None):
    m = inputs[0][0].shape[0] if m is None else m
    d_out = layers[-1][0].shape[1]
    in_specs = []
    args = []
    widths = []
    for a, off, w in inputs:
        in_specs.append(
            pl.BlockSpec((tile, a.shape[1]),
                         lambda i, off=off: (i + off, 0)))
        args.append(a)
        widths.append(w)
    for w, b in layers:
        in_specs.append(pl.BlockSpec(w.shape, lambda i: (0, 0)))
        in_specs.append(pl.BlockSpec((1, b.shape[0]), lambda i: (0, 0)))
        args.append(w)
        args.append(b.reshape(1, -1))
    body = functools.partial(_mlp_body, n_in=len(inputs), acts=tuple(acts),
                             widths=tuple(widths))
    return pl.pallas_call(
        body,
        grid=(m // tile,),
        in_specs=in_specs,
        out_specs=pl.BlockSpec((tile, d_out), lambda i: (i, 0)),
        out_shape=jax.ShapeDtypeStruct((m, d_out), jnp.float32),
    )(*args)


def _set2set(p, feat, seg, num_seg, dim, steps=3):
    q_star = jnp.zeros((num_seg, 2 * dim), dtype=feat.dtype)
    h = jnp.zeros((num_seg, dim), dtype=feat.dtype)
    c = jnp.zeros((num_seg, dim), dtype=feat.dtype)
    for _ in range(steps):
        gates = q_star @ p["W_ih"].T + p["b_ih"] + h @ p["W_hh"].T + p["b_hh"]
        i, f, g, o = jnp.split(gates, 4, axis=-1)
        i = jax.nn.sigmoid(i)
        f = jax.nn.sigmoid(f)
        g = jnp.tanh(g)
        o = jax.nn.sigmoid(o)
        c = f * c + i * g
        h = o * jnp.tanh(c)
        q = h
        e = jnp.sum(feat * q[seg], axis=-1)
        emax = jax.ops.segment_max(e, seg, num_segments=num_seg)
        emax = jnp.where(jnp.isfinite(emax), emax, 0.0)
        ee = jnp.exp(e - emax[seg])
        denom = jax.ops.segment_sum(ee, seg, num_segments=num_seg)
        a = ee / (denom[seg] + 1e-16)
        r = jax.ops.segment_sum(a[:, None] * feat, seg, num_segments=num_seg)
        q_star = jnp.concatenate([q, r], axis=-1)
    return q_star


def kernel(x, edge_index, edge_attr, batch, params):
    n_nodes = x.shape[0]
    n_edges = edge_attr.shape[0]
    depth = len(params["blocks"])
    src, dst = edge_index[0], edge_index[1]

    h_n = _mlp([(x, 0, None)],
               [(params["embed_nodes"]["W"], params["embed_nodes"]["b"])],
               ["none"], tile=2000)
    h_e = _mlp([(edge_attr, 0, None)],
               [(params["embed_edges"]["W"], params["embed_edges"]["b"])],
               ["none"], tile=2000)

    idx_both = jnp.concatenate([src, dst], axis=0)
    tile = 2000
    dst_off = n_edges // tile

    for i in range(depth):
        dn = params["dense_nodes"][i]
        h_n = _mlp([(h_n, 0, None)], [(dn["l1"]["W"], dn["l1"]["b"]),
                                      (dn["l2"]["W"], dn["l2"]["b"])],
                   ["relu", "none"], tile=tile)
        de = params["dense_edges"][i]
        h_e = _mlp([(h_e, 0, None)], [(de["l1"]["W"], de["l1"]["b"]),
                                      (de["l2"]["W"], de["l2"]["b"])],
                   ["none", "none"], tile=tile)

        gathered = _sc_gather(h_n, idx_both, chunk=1000)

        be = params["blocks"][i]["edge"]
        h_e = _mlp([(gathered, 0, _HID), (gathered, dst_off, _HID),
                    (h_e, 0, None)],
                   [(be[0]["W"], be[0]["b"]), (be[1]["W"], be[1]["b"]),
                    (be[2]["W"], be[2]["b"])],
                   ["softplus", "softplus", "none"], tile=tile, m=n_edges)

        agg = jax.ops.segment_sum(h_e, dst, num_segments=n_nodes)

        bn = params["blocks"][i]["node"]
        h_n = _mlp([(h_n, 0, None), (agg, 0, None)],
                   [(bn[0]["W"], bn[0]["b"]), (bn[1]["W"], bn[1]["b"]),
                    (bn[2]["W"], bn[2]["b"])],
                   ["softplus", "softplus", "none"], tile=tile)

    hn_r = _set2set(params["s2s_n"], h_n, batch, _NUM_GRAPHS, _HID)
    he_r = _set2set(params["s2s_e"], h_e, batch[src], _NUM_GRAPHS, _HID)

    out = jnp.concatenate([hn_r, he_r], axis=1)
    mp0, mp1 = params["mlp"]
    out = _mlp([(out, 0, None)],
               [(mp0["W"], mp0["b"]), (mp1["W"], mp1["b"]),
                (params["out"]["W"], params["out"]["b"])],
               ["relu", "relu", "none"], tile=_NUM_GRAPHS)
    return out


# speedup vs baseline: 1.9249x; 1.8753x over previous
import functools

import jax
import jax.numpy as jnp
from jax import lax
from jax.experimental import pallas as pl
from jax.experimental.pallas import tpu as pltpu
from jax.experimental.pallas import tpu_sc as plsc

_HID = 32
_NUM_GRAPHS = 64


def _sc_gather(table, idx, chunk):
    info = plsc.get_sparse_core_info()
    nc, ns = info.num_cores, info.num_subcores
    nw = nc * ns
    n, d = table.shape
    table128 = jnp.pad(table, ((0, 0), (0, 128 - d)))
    e = idx.shape[0]
    b_w = e // nw
    n_ch = b_w // chunk
    mesh = plsc.VectorSubcoreMesh(core_axis_name="c", subcore_axis_name="s")

    @functools.partial(
        pl.kernel,
        mesh=mesh,
        out_type=jax.ShapeDtypeStruct((e, 128), jnp.float32),
        scratch_types=[
            pltpu.VMEM((chunk,), jnp.int32),
            pltpu.VMEM((chunk, 128), jnp.float32),
            pltpu.SemaphoreType.DMA,
        ],
    )
    def gather_kernel(table_hbm, idx_hbm, out_hbm, idx_v, rows_v, sem):
        wid = lax.axis_index("s") * nc + lax.axis_index("c")
        base = wid * b_w
        for j in range(n_ch):
            off = base + j * chunk
            pltpu.sync_copy(idx_hbm.at[pl.ds(off, chunk)], idx_v)
            pltpu.async_copy(table_hbm.at[idx_v], rows_v, sem).wait()
            pltpu.sync_copy(rows_v, out_hbm.at[pl.ds(off, chunk)])

    return gather_kernel(table128, idx)


def _softplus(x):
    return jnp.maximum(x, 0.0) + jnp.log1p(jnp.exp(-jnp.abs(x)))


def _act(x, kind):
    if kind == "relu":
        return jnp.maximum(x, 0.0)
    if kind == "softplus":
        return _softplus(x)
    return x


def _mlp_body(*refs, n_in, acts, widths):
    in_refs = refs[:n_in]
    w_refs = refs[n_in:-1]
    o_ref = refs[-1]
    parts = []
    for r, w in zip(in_refs, widths):
        v = r[...]
        if w is not None and w < v.shape[1]:
            v = v[:, :w]
        parts.append(v)
    h = parts[0] if n_in == 1 else jnp.concatenate(parts, axis=1)
    n_layer = len(w_refs) // 2
    for li in range(n_layer):
        w = w_refs[2 * li][...]
        b = w_refs[2 * li + 1][...]
        h = jnp.dot(h, w, preferred_element_type=jnp.float32) + b
        h = _act(h, acts[li])
    o_ref[...] = h


def _mlp(inputs, layers, acts, tile, m=None):
    m = inputs[0][0].shape[0] if m is None else m
    d_out = layers[-1][0].shape[1]
    in_specs = []
    args = []
    widths = []
    for a, off, w in inputs:
        in_specs.append(
            pl.BlockSpec((tile, a.shape[1]),
                         lambda i, off=off: (i + off, 0)))
        args.append(a)
        widths.append(w)
    for w, b in layers:
        in_specs.append(pl.BlockSpec(w.shape, lambda i: (0, 0)))
        in_specs.append(pl.BlockSpec((1, b.shape[0]), lambda i: (0, 0)))
        args.append(w)
        args.append(b.reshape(1, -1))
    body = functools.partial(_mlp_body, n_in=len(inputs), acts=tuple(acts),
                             widths=tuple(widths))
    return pl.pallas_call(
        body,
        grid=(m // tile,),
        in_specs=in_specs,
        out_specs=pl.BlockSpec((tile, d_out), lambda i: (i, 0)),
        out_shape=jax.ShapeDtypeStruct((m, d_out), jnp.float32),
    )(*args)


def _s2s_a_body(feat_ref, q_ref, seg_ref, emax_ref, *, ns):
    i = pl.program_id(0)

    @pl.when(i == 0)
    def _():
        emax_ref[...] = jnp.full_like(emax_ref, -jnp.inf)

    seg = seg_ref[...]
    lanes = lax.broadcasted_iota(jnp.int32, (1, ns), 1)
    hit = seg == lanes
    oh = hit.astype(jnp.float32)
    qe = jnp.dot(oh, q_ref[...], preferred_element_type=jnp.float32,
                 precision=lax.Precision.HIGHEST)
    e = jnp.sum(feat_ref[...] * qe, axis=1, keepdims=True)
    masked = jnp.where(hit, e, -jnp.inf)
    emax_ref[...] = jnp.maximum(emax_ref[...],
                                jnp.max(masked, axis=0, keepdims=True))


def _s2s_b_body(feat_ref, q_ref, seg_ref, emax_ref, denom_ref, rnum_ref,
                *, ns):
    i = pl.program_id(0)

    @pl.when(i == 0)
    def _():
        denom_ref[...] = jnp.zeros_like(denom_ref)
        rnum_ref[...] = jnp.zeros_like(rnum_ref)

    seg = seg_ref[...]
    lanes = lax.broadcasted_iota(jnp.int32, (1, ns), 1)
    hit = seg == lanes
    oh = hit.astype(jnp.float32)
    feat = feat_ref[...]
    qe = jnp.dot(oh, q_ref[...], preferred_element_type=jnp.float32,
                 precision=lax.Precision.HIGHEST)
    e = jnp.sum(feat * qe, axis=1, keepdims=True)
    em = jnp.sum(oh * emax_ref[...], axis=1, keepdims=True)
    ee = jnp.exp(e - em)
    denom_ref[...] += jnp.sum(oh * ee, axis=0, keepdims=True)
    rnum_ref[...] += lax.dot_general(
        oh, feat * ee, (((0,), (0,)), ((), ())),
        preferred_element_type=jnp.float32,
        precision=lax.Precision.HIGHEST)


def _s2s_softmax(feat, q, seg2d, ns, tile):
    m, d = feat.shape
    grid = (m // tile,)
    feat_spec = pl.BlockSpec((tile, d), lambda i: (i, 0))
    q_spec = pl.BlockSpec((ns, d), lambda i: (0, 0))
    seg_spec = pl.BlockSpec((tile, 1), lambda i: (i, 0))
    vec_spec = pl.BlockSpec((1, ns), lambda i: (0, 0))
    emax = pl.pallas_call(
        functools.partial(_s2s_a_body, ns=ns),
        grid=grid,
        in_specs=[feat_spec, q_spec, seg_spec],
        out_specs=vec_spec,
        out_shape=jax.ShapeDtypeStruct((1, ns), jnp.float32),
    )(feat, q, seg2d)
    emax = jnp.where(jnp.isfinite(emax), emax, 0.0)
    denom, rnum = pl.pallas_call(
        functools.partial(_s2s_b_body, ns=ns),
        grid=grid,
        in_specs=[feat_spec, q_spec, seg_spec, vec_spec],
        out_specs=[vec_spec, pl.BlockSpec((ns, d), lambda i: (0, 0))],
        out_shape=[jax.ShapeDtypeStruct((1, ns), jnp.float32),
                   jax.ShapeDtypeStruct((ns, d), jnp.float32)],
    )(feat, q, seg2d, emax)
    return rnum / (denom.reshape(ns, 1) + 1e-16)


def _set2set(p, feat, seg2d, num_seg, dim, tile, steps=3):
    q_star = jnp.zeros((num_seg, 2 * dim), dtype=feat.dtype)
    h = jnp.zeros((num_seg, dim), dtype=feat.dtype)
    c = jnp.zeros((num_seg, dim), dtype=feat.dtype)
    for _ in range(steps):
        gates = q_star @ p["W_ih"].T + p["b_ih"] + h @ p["W_hh"].T + p["b_hh"]
        i, f, g, o = jnp.split(gates, 4, axis=-1)
        i = jax.nn.sigmoid(i)
        f = jax.nn.sigmoid(f)
        g = jnp.tanh(g)
        o = jax.nn.sigmoid(o)
        c = f * c + i * g
        h = o * jnp.tanh(c)
        q = h
        r = _s2s_softmax(feat, q, seg2d, num_seg, tile)
        q_star = jnp.concatenate([q, r], axis=-1)
    return q_star


def kernel(x, edge_index, edge_attr, batch, params):
    n_nodes = x.shape[0]
    n_edges = edge_attr.shape[0]
    depth = len(params["blocks"])
    src, dst = edge_index[0], edge_index[1]

    h_n = _mlp([(x, 0, None)],
               [(params["embed_nodes"]["W"], params["embed_nodes"]["b"])],
               ["none"], tile=2000)
    h_e = _mlp([(edge_attr, 0, None)],
               [(params["embed_edges"]["W"], params["embed_edges"]["b"])],
               ["none"], tile=2000)

    idx_both = jnp.concatenate([src, dst], axis=0)
    tile = 2000
    dst_off = n_edges // tile

    for i in range(depth):
        dn = params["dense_nodes"][i]
        h_n = _mlp([(h_n, 0, None)], [(dn["l1"]["W"], dn["l1"]["b"]),
                                      (dn["l2"]["W"], dn["l2"]["b"])],
                   ["relu", "none"], tile=tile)
        de = params["dense_edges"][i]
        h_e = _mlp([(h_e, 0, None)], [(de["l1"]["W"], de["l1"]["b"]),
                                      (de["l2"]["W"], de["l2"]["b"])],
                   ["none", "none"], tile=tile)

        gathered = _sc_gather(h_n, idx_both, chunk=1000)

        be = params["blocks"][i]["edge"]
        h_e = _mlp([(gathered, 0, _HID), (gathered, dst_off, _HID),
                    (h_e, 0, None)],
                   [(be[0]["W"], be[0]["b"]), (be[1]["W"], be[1]["b"]),
                    (be[2]["W"], be[2]["b"])],
                   ["softplus", "softplus", "none"], tile=tile, m=n_edges)

        agg = jax.ops.segment_sum(h_e, dst, num_segments=n_nodes)

        bn = params["blocks"][i]["node"]
        h_n = _mlp([(h_n, 0, None), (agg, 0, None)],
                   [(bn[0]["W"], bn[0]["b"]), (bn[1]["W"], bn[1]["b"]),
                    (bn[2]["W"], bn[2]["b"])],
                   ["softplus", "softplus", "none"], tile=tile)

    hn_r = _set2set(params["s2s_n"], h_n, batch.reshape(-1, 1),
                    _NUM_GRAPHS, _HID, tile=2000)
    he_r = _set2set(params["s2s_e"], h_e, batch[src].reshape(-1, 1),
                    _NUM_GRAPHS, _HID, tile=2000)

    out = jnp.concatenate([hn_r, he_r], axis=1)
    mp0, mp1 = params["mlp"]
    out = _mlp([(out, 0, None)],
               [(mp0["W"], mp0["b"]), (mp1["W"], mp1["b"]),
                (params["out"]["W"], params["out"]["b"])],
               ["relu", "relu", "none"], tile=_NUM_GRAPHS)
    return out
